# Initial kernel scaffold; baseline (speedup 1.0000x reference)
import jax
import jax.numpy as jnp
from jax.experimental import pallas as pl


def kernel(x, track_idxs, y):
    raise NotImplementedError("write your pallas kernel here")


# speedup vs baseline: 22.4386x; 22.4386x over previous
import jax
import jax.numpy as jnp
from jax.experimental import pallas as pl

_TEMP = 0.3
_EPS = 1e-09
_EPS2 = 1e-10
_N = 4096
_V = 64
_Q = 2
_D = 64
_J = _V * _Q


def _loss_kernel(x_ref, ti_ref, y2_ref, out_ref):
    x = x_ref[...]
    ti = ti_ref[...]
    y2 = y2_ref[...]

    vals_i = jax.lax.broadcasted_iota(jnp.int32, (1, _V), 1)
    onehot = (ti == vals_i).astype(jnp.float32)
    pf = (jnp.sum(onehot, axis=0, keepdims=True) > 0.0).astype(jnp.float32)
    u_cnt = jnp.sum(pf)

    iota_r = jax.lax.broadcasted_iota(jnp.int32, (_V, _V), 0)
    iota_c = jax.lax.broadcasted_iota(jnp.int32, (_V, _V), 1)
    lt = (iota_r < iota_c).astype(jnp.float32)
    rank = jax.lax.dot_general(pf, lt, (((1,), (0,)), ((), ())),
                               preferred_element_type=jnp.float32)

    rowrank = jnp.sum(rank * onehot, axis=1, keepdims=True)

    jj = jax.lax.broadcasted_iota(jnp.int32, (_J, 1), 0)
    s_j = (jj // 2).astype(jnp.float32)
    sel = ((rank == s_j).astype(jnp.float32)) * pf
    valsf = vals_i.astype(jnp.float32)
    u_row = jnp.sum(sel * valsf, axis=1, keepdims=True)
    u_row = jnp.where(s_j < u_cnt, u_row, jnp.float32(_V - 1))
    par = (jj - 2 * (jj // 2)).astype(jnp.float32)

    kk = jax.lax.broadcasted_iota(jnp.int32, (1, _J), 1).astype(jnp.float32)
    g = (kk == (2.0 * u_row + par)).astype(jnp.float32)
    y_rows = jax.lax.dot_general(g, y2, (((1,), (0,)), ((), ())),
                                 preferred_element_type=jnp.float32)

    logits = jax.lax.dot_general(x, y_rows, (((1,), (1,)), ((), ())),
                                 preferred_element_type=jnp.float32)
    e = jnp.exp(logits * (1.0 / _TEMP))

    valid = (kk < 2.0 * u_cnt)
    jmod = kk - u_cnt * jnp.floor(kk / u_cnt)
    match = (rowrank == jmod) & valid
    num = jnp.sum(e * match.astype(jnp.float32), axis=(0, 1), keepdims=True)
    total = jnp.sum(e * valid.astype(jnp.float32), axis=(0, 1), keepdims=True)
    den = total - num
    out_ref[...] = -jnp.log(num / (den + _EPS) + _EPS2)


@jax.jit
def kernel(x, track_idxs, y):
    ti = track_idxs.astype(jnp.int32).reshape(_N, 1)
    y2 = y.reshape(_J, _D)
    out = pl.pallas_call(
        _loss_kernel,
        out_shape=jax.ShapeDtypeStruct((1, 1), jnp.float32),
    )(x, ti, y2)
    return out.reshape(1)
